# Initial kernel scaffold; baseline (speedup 1.0000x reference)
import functools

import jax
import jax.numpy as jnp
from jax import lax
from jax.experimental import pallas as pl
from jax.experimental.pallas import tpu as pltpu
from jax.experimental.pallas import tpu_sc as plsc

VOCAB = 1000000
EMBED_DIM = 32
B = 16384
L = 50
BT = B * L

NUM_CORES = 2
NUM_SUBCORES = 16
NW = NUM_CORES * NUM_SUBCORES
B_PER_W = BT // NW
CHUNK = 1600
NCHUNK = B_PER_W // CHUNK


def _make_kernel():
    mesh = plsc.VectorSubcoreMesh(core_axis_name="c", subcore_axis_name="s")

    @functools.partial(
        pl.kernel,
        mesh=mesh,
        out_type=jax.ShapeDtypeStruct((BT, EMBED_DIM), jnp.float32),
        scratch_types=[
            pltpu.VMEM((CHUNK,), jnp.int32),
            pltpu.VMEM((CHUNK, EMBED_DIM), jnp.float32),
            pltpu.SemaphoreType.DMA,
        ],
    )
    def gather_kernel(idx_hbm, table_hbm, out_hbm, idx_v, rows_v, sem):
        wid = lax.axis_index("s") * NUM_CORES + lax.axis_index("c")
        wbase = wid * B_PER_W

        def body(j, carry):
            base = wbase + j * CHUNK
            pltpu.sync_copy(idx_hbm.at[pl.ds(base, CHUNK)], idx_v)
            pltpu.async_copy(table_hbm.at[idx_v], rows_v, sem).wait()
            pltpu.sync_copy(rows_v, out_hbm.at[pl.ds(base, CHUNK)])
            return carry

        lax.fori_loop(0, NCHUNK, body, 0)

    return gather_kernel


_gather = _make_kernel()


def kernel(words, table):
    flat = words.reshape(BT)
    out = _gather(flat, table)
    return out.reshape(B, L, EMBED_DIM)


# speedup vs baseline: 1.1031x; 1.1031x over previous
import functools

import jax
import jax.numpy as jnp
from jax import lax
from jax.experimental import pallas as pl
from jax.experimental.pallas import tpu as pltpu
from jax.experimental.pallas import tpu_sc as plsc

VOCAB = 1000000
EMBED_DIM = 32
B = 16384
L = 50
BT = B * L

NUM_CORES = 2
NUM_SUBCORES = 16
NW = NUM_CORES * NUM_SUBCORES
B_PER_W = BT // NW
CHUNK = 1600
NCHUNK = B_PER_W // CHUNK


def _make_kernel():
    mesh = plsc.VectorSubcoreMesh(core_axis_name="c", subcore_axis_name="s")

    @functools.partial(
        pl.kernel,
        mesh=mesh,
        out_type=jax.ShapeDtypeStruct((BT, EMBED_DIM), jnp.float32),
        scratch_types=[
            pltpu.VMEM((CHUNK,), jnp.int32),
            pltpu.VMEM((CHUNK, EMBED_DIM), jnp.float32),
            pltpu.SemaphoreType.DMA,
        ],
        compiler_params=pltpu.CompilerParams(use_tc_tiling_on_sc=False),
    )
    def gather_kernel(idx_hbm, table_hbm, out_hbm, idx_v, rows_v, sem):
        wid = lax.axis_index("s") * NUM_CORES + lax.axis_index("c")
        wbase = wid * B_PER_W

        def body(j, carry):
            base = wbase + j * CHUNK
            pltpu.sync_copy(idx_hbm.at[pl.ds(base, CHUNK)], idx_v)
            pltpu.async_copy(table_hbm.at[idx_v], rows_v, sem).wait()
            pltpu.sync_copy(rows_v, out_hbm.at[pl.ds(base, CHUNK)])
            return carry

        lax.fori_loop(0, NCHUNK, body, 0)

    return gather_kernel


_gather = _make_kernel()


def kernel(words, table):
    flat = words.reshape(BT)
    out = _gather(flat, table)
    return out.reshape(B, L, EMBED_DIM)


# speedup vs baseline: 1.1139x; 1.0097x over previous
import functools

import jax
import jax.numpy as jnp
from jax import lax
from jax.experimental import pallas as pl
from jax.experimental.pallas import tpu as pltpu
from jax.experimental.pallas import tpu_sc as plsc

VOCAB = 1000000
EMBED_DIM = 32
B = 16384
L = 50
BT = B * L

NUM_CORES = 2
NUM_SUBCORES = 16
NW = NUM_CORES * NUM_SUBCORES
B_PER_W = BT // NW
CHUNK = 800
NCHUNK = B_PER_W // CHUNK
NBUF = 4


def _make_kernel():
    mesh = plsc.VectorSubcoreMesh(core_axis_name="c", subcore_axis_name="s")

    @functools.partial(
        pl.kernel,
        mesh=mesh,
        out_type=jax.ShapeDtypeStruct((BT, EMBED_DIM), jnp.float32),
        scratch_types=[
            [pltpu.VMEM((CHUNK,), jnp.int32) for _ in range(NBUF)],
            [pltpu.VMEM((CHUNK, EMBED_DIM), jnp.float32) for _ in range(NBUF)],
            [pltpu.SemaphoreType.DMA for _ in range(NBUF)],
            [pltpu.SemaphoreType.DMA for _ in range(NBUF)],
        ],
        compiler_params=pltpu.CompilerParams(use_tc_tiling_on_sc=False),
    )
    def gather_kernel(idx_hbm, table_hbm, out_hbm, idx_v, rows_v, gsem, wsem):
        wid = lax.axis_index("s") * NUM_CORES + lax.axis_index("c")
        wbase = wid * B_PER_W

        for p in range(NBUF):
            pltpu.sync_copy(idx_hbm.at[pl.ds(wbase + p * CHUNK, CHUNK)], idx_v[p])
            pltpu.async_copy(table_hbm.at[idx_v[p]], rows_v[p], gsem[p])

        def body(j, carry):
            for p in range(NBUF):
                base = wbase + (j * NBUF + p) * CHUNK
                pltpu.make_async_copy(table_hbm.at[idx_v[p]], rows_v[p],
                                      gsem[p]).wait()
                pltpu.async_copy(rows_v[p], out_hbm.at[pl.ds(base, CHUNK)],
                                 wsem[p])
                nbase = base + NBUF * CHUNK
                pltpu.sync_copy(idx_hbm.at[pl.ds(nbase, CHUNK)], idx_v[p])
                pltpu.make_async_copy(rows_v[p], out_hbm.at[pl.ds(base, CHUNK)],
                                      wsem[p]).wait()
                pltpu.async_copy(table_hbm.at[idx_v[p]], rows_v[p], gsem[p])
            return carry

        lax.fori_loop(0, NCHUNK // NBUF - 1, body, 0)

        for p in range(NBUF):
            base = wbase + (NCHUNK - NBUF + p) * CHUNK
            pltpu.make_async_copy(table_hbm.at[idx_v[p]], rows_v[p],
                                  gsem[p]).wait()
            pltpu.async_copy(rows_v[p], out_hbm.at[pl.ds(base, CHUNK)], wsem[p])
        for p in range(NBUF):
            base = wbase + (NCHUNK - NBUF + p) * CHUNK
            pltpu.make_async_copy(rows_v[p], out_hbm.at[pl.ds(base, CHUNK)],
                                  wsem[p]).wait()

    return gather_kernel


_gather = _make_kernel()


def kernel(words, table):
    flat = words.reshape(BT)
    out = _gather(flat, table)
    return out.reshape(B, L, EMBED_DIM)


# speedup vs baseline: 1.8055x; 1.6209x over previous
import functools

import jax
import jax.numpy as jnp
from jax import lax
from jax.experimental import pallas as pl
from jax.experimental.pallas import tpu as pltpu
from jax.experimental.pallas import tpu_sc as plsc

VOCAB = 1000000
EMBED_DIM = 32
B = 16384
L = 50
BT = B * L

NUM_CORES = 2
NUM_SUBCORES = 16
NW = NUM_CORES * NUM_SUBCORES
B_PER_W = BT // NW
CHUNK = 800
CB = CHUNK // L
NCHUNK = B_PER_W // CHUNK
NBUF = 4


def _make_kernel():
    mesh = plsc.VectorSubcoreMesh(core_axis_name="c", subcore_axis_name="s")

    @functools.partial(
        pl.kernel,
        mesh=mesh,
        out_type=jax.ShapeDtypeStruct((B, L, EMBED_DIM), jnp.float32),
        scratch_types=[
            [pltpu.VMEM((CB, L), jnp.int32) for _ in range(NBUF)],
            [pltpu.VMEM((CB, L, EMBED_DIM), jnp.float32) for _ in range(NBUF)],
            [pltpu.SemaphoreType.DMA for _ in range(NBUF)],
            [pltpu.SemaphoreType.DMA for _ in range(NBUF)],
        ],
        compiler_params=pltpu.CompilerParams(use_tc_tiling_on_sc=False),
    )
    def gather_kernel(idx_hbm, table_hbm, out_hbm, idx_v, rows_v, gsem, wsem):
        wid = lax.axis_index("s") * NUM_CORES + lax.axis_index("c")
        wb = wid * (B_PER_W // L)

        def wb_copy(p, bb):
            return pltpu.make_async_copy(
                rows_v[p], out_hbm.at[pl.ds(bb, CB)], wsem[p])

        def start_gathers(p):
            def gbody(k, carry):
                pltpu.async_copy(
                    table_hbm.at[idx_v[p].at[k]],
                    rows_v[p].at[k], gsem[p])
                return carry
            lax.fori_loop(0, CB, gbody, 0)

        def wait_gathers(p, bb):
            pltpu.make_async_copy(out_hbm.at[pl.ds(bb, CB)], rows_v[p],
                                  gsem[p]).wait()

        for p in range(NBUF):
            pltpu.sync_copy(idx_hbm.at[pl.ds(wb + p * CB, CB)], idx_v[p])
            start_gathers(p)

        def body(j, carry):
            for p in range(NBUF):
                c = j * NBUF + p
                bb = wb + c * CB
                wait_gathers(p, bb)
                wb_copy(p, bb).start()
                pltpu.sync_copy(idx_hbm.at[pl.ds(bb + NBUF * CB, CB)],
                                idx_v[p])
                wb_copy(p, bb).wait()
                start_gathers(p)
            return carry

        lax.fori_loop(0, NCHUNK // NBUF - 1, body, 0)

        for p in range(NBUF):
            c = NCHUNK - NBUF + p
            wait_gathers(p, wb + c * CB)
            wb_copy(p, wb + c * CB).start()
        for p in range(NBUF):
            c = NCHUNK - NBUF + p
            wb_copy(p, wb + c * CB).wait()

    return gather_kernel


_gather = _make_kernel()


def kernel(words, table):
    return _gather(words, table)
